# Initial kernel scaffold; baseline (speedup 1.0000x reference)
import jax
import jax.numpy as jnp
from jax.experimental import pallas as pl


def kernel(query, positive, negative, other_neg):
    raise NotImplementedError("write your pallas kernel here")


# speedup vs baseline: 3.3425x; 3.3425x over previous
import functools

import jax
import jax.numpy as jnp
from jax.experimental import pallas as pl
from jax.experimental.pallas import tpu as pltpu


def _count_kernel(q_ref, p_ref, n_ref, on_ref, o_ref, neg_scr, *, NN):
    eps = 1e-12
    q = q_ref[0]
    on = on_ref[0]

    qv = q - on
    qn = qv / jnp.maximum(
        jnp.sqrt(jnp.sum(qv * qv, axis=1, keepdims=True)), eps)

    pv = p_ref[0] - on
    pn = pv / jnp.maximum(
        jnp.sqrt(jnp.sum(pv * pv, axis=1, keepdims=True)), eps)

    nv = n_ref[0] - on
    nn_ = nv / jnp.maximum(
        jnp.sqrt(jnp.sum(nv * nv, axis=1, keepdims=True)), eps)

    dims = (((1,), (1,)), ((), ()))
    cposT = 2.0 - 2.0 * jax.lax.dot_general(
        pn, qn, dims, preferred_element_type=jnp.float32)
    neg_scr[...] = 2.0 - 2.0 * jax.lax.dot_general(
        nn_, qn, dims, preferred_element_type=jnp.float32)

    def body(j, acc):
        nrow = neg_scr[pl.ds(j, 1), :]
        return acc + (nrow < cposT).astype(jnp.float32)

    acc = jax.lax.fori_loop(
        0, NN, body, jnp.zeros(cposT.shape, jnp.float32))
    o_ref[0, 0, :] = jnp.sum(acc, axis=0)


def kernel(query, positive, negative, other_neg):
    B, N, D = query.shape
    RC = min(128, N)

    nblk = N // RC
    grid = (B, nblk)
    u1 = pl.pallas_call(
        functools.partial(_count_kernel, NN=N),
        grid=grid,
        in_specs=[
            pl.BlockSpec((1, RC, D), lambda b, r: (b, r, 0)),
            pl.BlockSpec((1, N, D), lambda b, r: (b, 0, 0)),
            pl.BlockSpec((1, N, D), lambda b, r: (b, 0, 0)),
            pl.BlockSpec((1, 1, D), lambda b, r: (b, 0, 0)),
        ],
        out_specs=pl.BlockSpec(
            (1, 1, RC), lambda b, r, nblk=nblk: (b * nblk + r, 0, 0)),
        out_shape=jax.ShapeDtypeStruct((B * nblk, 1, RC), jnp.float32),
        scratch_shapes=[pltpu.VMEM((N, RC), jnp.float32)],
    )(query, positive, negative, other_neg)
    u1 = u1.reshape(B, N)

    n1 = n2 = float(N)
    mu = n1 * n2 / 2.0
    sigma = jnp.sqrt(n1 * n2 * (n1 + n2 + 1.0) / 12.0)
    u2 = n1 * n2 - u1
    z1 = (u1 - mu) / sigma
    z2 = (u2 - mu) / sigma
    inv_sqrt2 = 1.0 / jnp.sqrt(jnp.float32(2.0))
    p1 = 0.5 * (1.0 + jax.scipy.special.erf(z1 * inv_sqrt2))
    p2 = 0.5 * (1.0 + jax.scipy.special.erf(z2 * inv_sqrt2))
    loss = jnp.maximum(1.0 - p2 - 0.05, 0.0) + jnp.maximum(p1 - 0.05, 0.0)
    return jnp.mean(loss)


# speedup vs baseline: 10.6245x; 3.1786x over previous
import functools

import jax
import jax.numpy as jnp
from jax.experimental import pallas as pl
from jax.experimental.pallas import tpu as pltpu


def _count_kernel(q_ref, p_ref, n_ref, on_ref, o_ref, neg_scr, pos_scr,
                  *, NN):
    eps = 1e-12
    q = q_ref[0]
    on = on_ref[0]

    qv = q - on
    qn = qv / jnp.maximum(
        jnp.sqrt(jnp.sum(qv * qv, axis=1, keepdims=True)), eps)

    pv = p_ref[0] - on
    pn = pv / jnp.maximum(
        jnp.sqrt(jnp.sum(pv * pv, axis=1, keepdims=True)), eps)

    nv = n_ref[0] - on
    nn_ = nv / jnp.maximum(
        jnp.sqrt(jnp.sum(nv * nv, axis=1, keepdims=True)), eps)

    dims = (((1,), (1,)), ((), ()))
    pos_scr[...] = 2.0 - 2.0 * jax.lax.dot_general(
        pn, qn, dims, preferred_element_type=jnp.float32)
    neg_scr[...] = 2.0 - 2.0 * jax.lax.dot_general(
        nn_, qn, dims, preferred_element_type=jnp.float32)

    RC = pos_scr.shape[1]
    IC = 64
    JU = 4
    total = jnp.zeros((1, RC), jnp.float32)
    for blk in range(NN // IC):
        cpos_blk = pos_scr[blk * IC:(blk + 1) * IC, :]

        def jbody(j, acc, cpos_blk=cpos_blk):
            for u in range(JU):
                nrow = neg_scr[pl.ds(j * JU + u, 1), :]
                acc = acc + (nrow < cpos_blk).astype(jnp.float32)
            return acc

        acc = jax.lax.fori_loop(
            0, NN // JU, jbody, jnp.zeros((IC, RC), jnp.float32))
        total = total + jnp.sum(acc, axis=0, keepdims=True)
    o_ref[0, 0, :] = total[0]


def kernel(query, positive, negative, other_neg):
    B, N, D = query.shape
    RC = min(128, N)

    nblk = N // RC
    grid = (B, nblk)
    u1 = pl.pallas_call(
        functools.partial(_count_kernel, NN=N),
        grid=grid,
        in_specs=[
            pl.BlockSpec((1, RC, D), lambda b, r: (b, r, 0)),
            pl.BlockSpec((1, N, D), lambda b, r: (b, 0, 0)),
            pl.BlockSpec((1, N, D), lambda b, r: (b, 0, 0)),
            pl.BlockSpec((1, 1, D), lambda b, r: (b, 0, 0)),
        ],
        out_specs=pl.BlockSpec(
            (1, 1, RC), lambda b, r, nblk=nblk: (b * nblk + r, 0, 0)),
        out_shape=jax.ShapeDtypeStruct((B * nblk, 1, RC), jnp.float32),
        scratch_shapes=[pltpu.VMEM((N, RC), jnp.float32),
                        pltpu.VMEM((N, RC), jnp.float32)],
    )(query, positive, negative, other_neg)
    u1 = u1.reshape(B, N)

    n1 = n2 = float(N)
    mu = n1 * n2 / 2.0
    sigma = jnp.sqrt(n1 * n2 * (n1 + n2 + 1.0) / 12.0)
    u2 = n1 * n2 - u1
    z1 = (u1 - mu) / sigma
    z2 = (u2 - mu) / sigma
    inv_sqrt2 = 1.0 / jnp.sqrt(jnp.float32(2.0))
    p1 = 0.5 * (1.0 + jax.scipy.special.erf(z1 * inv_sqrt2))
    p2 = 0.5 * (1.0 + jax.scipy.special.erf(z2 * inv_sqrt2))
    loss = jnp.maximum(1.0 - p2 - 0.05, 0.0) + jnp.maximum(p1 - 0.05, 0.0)
    return jnp.mean(loss)


# speedup vs baseline: 26.1703x; 2.4632x over previous
import functools

import jax
import jax.numpy as jnp
from jax import lax
from jax.experimental import pallas as pl
from jax.experimental.pallas import tpu as pltpu
from jax.experimental.pallas import tpu_sc as plsc


def _cosim_kernel(q_ref, p_ref, n_ref, on_ref, cp_ref, cn_ref):
    eps = 1e-12
    q = q_ref[0]
    on = on_ref[0]
    qv = q - on
    qn = qv / jnp.maximum(
        jnp.sqrt(jnp.sum(qv * qv, axis=1, keepdims=True)), eps)
    pv = p_ref[0] - on
    pn = pv / jnp.maximum(
        jnp.sqrt(jnp.sum(pv * pv, axis=1, keepdims=True)), eps)
    nv = n_ref[0] - on
    nn_ = nv / jnp.maximum(
        jnp.sqrt(jnp.sum(nv * nv, axis=1, keepdims=True)), eps)
    dims = (((1,), (1,)), ((), ()))
    cp_ref[...] = 2.0 - 2.0 * lax.dot_general(
        qn, pn, dims, preferred_element_type=jnp.float32)
    cn_ref[...] = 2.0 - 2.0 * lax.dot_general(
        qn, nn_, dims, preferred_element_type=jnp.float32)


def _cosim(query, positive, negative, other_neg):
    B, N, D = query.shape
    RC = 256
    nblk = N // RC
    grid = (B, nblk)
    return pl.pallas_call(
        _cosim_kernel,
        grid=grid,
        in_specs=[
            pl.BlockSpec((1, RC, D), lambda b, r: (b, r, 0)),
            pl.BlockSpec((1, N, D), lambda b, r: (b, 0, 0)),
            pl.BlockSpec((1, N, D), lambda b, r: (b, 0, 0)),
            pl.BlockSpec((1, 1, D), lambda b, r: (b, 0, 0)),
        ],
        out_specs=[
            pl.BlockSpec((RC, N), lambda b, r, nblk=nblk: (b * nblk + r, 0)),
            pl.BlockSpec((RC, N), lambda b, r, nblk=nblk: (b * nblk + r, 0)),
        ],
        out_shape=[
            jax.ShapeDtypeStruct((B * N, N), jnp.float32),
            jax.ShapeDtypeStruct((B * N, N), jnp.float32),
        ],
    )(query, positive, negative, other_neg)


_V = 16


def _mkey(x):
    bi = lax.bitcast_convert_type(x, jnp.int32)
    sgn = lax.shift_right_arithmetic(bi, 31)
    sgnu = lax.bitcast_convert_type(sgn, jnp.uint32)
    xu = lax.bitcast_convert_type(x, jnp.uint32)
    return xu ^ (sgnu | jnp.uint32(0x80000000))


def _splat(x, dtype=None):
    v = jnp.full((_V,), x)
    return v if dtype is None else v.astype(dtype)


def _perm(v, idx):
    dnums = lax.GatherDimensionNumbers(
        offset_dims=(), collapsed_slice_dims=(0,), start_index_map=(0,))
    return lax.gather(
        v, idx.reshape(_V, 1), dnums, (1,),
        mode=lax.GatherScatterMode.PROMISE_IN_BOUNDS)


def _lane():
    return lax.broadcasted_iota(jnp.int32, (_V,), 0)


def _allreduce(v, op):
    lane = _lane()
    for step in (8, 4, 2, 1):
        v = op(v, _perm(v, lane ^ step))
    return v


def _occurrence(d1):
    lane = _lane()
    occ = jnp.zeros((_V,), jnp.int32)
    one = jnp.ones((_V,), jnp.int32)
    zero = jnp.zeros((_V,), jnp.int32)
    for s in range(1, _V):
        g = _perm(d1, jnp.maximum(lane - s, 0))
        eq = (g == d1) & (lane >= s)
        occ = occ + jnp.where(eq, one, zero)
    return occ


def _occ_rev(d1):
    lane = _lane()
    occ = jnp.zeros((_V,), jnp.int32)
    one = jnp.ones((_V,), jnp.int32)
    zero = jnp.zeros((_V,), jnp.int32)
    for s in range(1, _V):
        g = _perm(d1, jnp.minimum(lane + s, _V - 1))
        eq = (g == d1) & (lane <= _V - 1 - s)
        occ = occ + jnp.where(eq, one, zero)
    return occ


def _prefix_sum(v):
    lane = _lane()
    zero = jnp.zeros((_V,), v.dtype)
    for step in (1, 2, 4, 8):
        g = _perm(v, jnp.maximum(lane - step, 0))
        v = v + jnp.where(lane >= step, g, zero)
    return v


def _make_sc_count(BN, N, RPW):
    VN = N // _V
    mesh = plsc.VectorSubcoreMesh(core_axis_name="c", subcore_axis_name="s")

    @functools.partial(
        pl.kernel, mesh=mesh,
        compiler_params=pltpu.CompilerParams(needs_layout_passes=False),
        out_type=jax.ShapeDtypeStruct((BN,), jnp.float32),
        scratch_types=[
            pltpu.VMEM((N,), jnp.float32),
            pltpu.VMEM((N,), jnp.float32),
            pltpu.VMEM((N,), jnp.int32),
            pltpu.VMEM((N,), jnp.int32),
            pltpu.VMEM((N,), jnp.int32),
            pltpu.VMEM((N,), jnp.int32),
            pltpu.VMEM((N,), jnp.int32),
            pltpu.VMEM((N,), jnp.int32),
            pltpu.VMEM((N,), jnp.int32),
            pltpu.VMEM((RPW,), jnp.float32),
            pltpu.VMEM((_V,), jnp.int32),
        ],
    )
    def sc_count(cp_hbm, cn_hbm, out_hbm, bufP, bufG, keyG, d1G, occG,
                 ctr, ctr2, excl, sortedG, outv, tmpv):
        nc = 2
        wid = lax.axis_index("s") * nc + lax.axis_index("c")
        base_row = wid * RPW
        lane = lax.broadcasted_iota(jnp.int32, (_V,), 0)

        def row_body(r, _):
            row = base_row + r
            pltpu.sync_copy(cp_hbm.at[row], bufP)
            pltpu.sync_copy(cn_hbm.at[row], bufG)

            def mm(v, c):
                mn, mx = c
                a = bufP[pl.ds(v * _V, _V)]
                b = bufG[pl.ds(v * _V, _V)]
                mn = jnp.minimum(jnp.minimum(mn, a), b)
                mx = jnp.maximum(jnp.maximum(mx, a), b)
                return (mn, mx)

            mnv, mxv = lax.fori_loop(
                0, VN, mm,
                (_splat(jnp.float32(jnp.inf)), _splat(-jnp.float32(jnp.inf))))
            mn_v = _mkey(_allreduce(mnv, jnp.minimum))
            mx_v = _mkey(_allreduce(mxv, jnp.maximum))

            rng = (mx_v - mn_v).astype(jnp.float32)
            ebits = lax.shift_right_logical(
                lax.bitcast_convert_type(rng, jnp.int32), 23) & 255
            sh_v = jnp.maximum(ebits - 126 - 11, 0).astype(jnp.uint32)

            def dig(k):
                return lax.shift_right_logical(k - mn_v, sh_v).astype(
                    jnp.int32)

            def zz(v, _c):
                ctr[pl.ds(v * _V, _V)] = jnp.zeros((_V,), jnp.int32)
                ctr2[pl.ds(v * _V, _V)] = jnp.zeros((_V,), jnp.int32)
                return 0

            lax.fori_loop(0, VN, zz, 0)

            def pa(v, _c):
                sl = pl.ds(v * _V, _V)
                k = _mkey(bufG[sl])
                d1 = dig(k)
                keyG[sl] = lax.bitcast_convert_type(k, jnp.int32)
                d1G[sl] = d1
                occ = _occurrence(d1)
                occG[sl] = occ
                b0 = plsc.load_gather(ctr, [d1])
                plsc.store_scatter(ctr, [d1], b0 + occ + 1,
                                   mask=_occ_rev(d1) == 0)
                return 0

            lax.fori_loop(0, VN, pa, 0)

            def cs(v, carry):
                sl = pl.ds(v * _V, _V)
                hv = ctr[sl]
                excl[sl] = _prefix_sum(hv) - hv + carry
                return carry + _allreduce(hv, jnp.add)

            lax.fori_loop(0, VN, cs, jnp.zeros((_V,), jnp.int32))

            def pb(v, _c):
                sl = pl.ds(v * _V, _V)
                d1 = d1G[sl]
                occ = occG[sl]
                k = keyG[sl]
                b0 = plsc.load_gather(ctr2, [d1])
                e0 = plsc.load_gather(excl, [d1])
                plsc.store_scatter(sortedG, [e0 + b0 + occ], k)
                plsc.store_scatter(ctr2, [d1], b0 + occ + 1,
                                   mask=_occ_rev(d1) == 0)
                return 0

            lax.fori_loop(0, VN, pb, 0)

            def pc(v, acc):
                sl = pl.ds(v * _V, _V)
                k = _mkey(bufP[sl])
                d1 = dig(k)
                e0 = plsc.load_gather(excl, [d1])
                cnt = plsc.load_gather(ctr, [d1])
                acc = acc + e0
                tmax = _allreduce(cnt, jnp.maximum)[0]
                cm1 = jnp.maximum(cnt - 1, 0)

                def rb(t, a2):
                    tv = _splat(t)
                    g = plsc.load_gather(
                        sortedG, [e0 + jnp.minimum(tv, cm1)])
                    gu = lax.bitcast_convert_type(g, jnp.uint32)
                    hit = (gu < k) & (tv < cnt)
                    return a2 + jnp.where(hit, 1, 0).astype(jnp.int32)

                acc = acc + lax.fori_loop(
                    0, tmax, rb, jnp.zeros((_V,), jnp.int32))
                return acc

            accv = lax.fori_loop(0, VN, pc, jnp.zeros((_V,), jnp.int32))
            u1v = _allreduce(accv, jnp.add).astype(jnp.float32)
            plsc.store_scatter(outv, [_splat(r).astype(jnp.int32)],
                               u1v, mask=lane == 0)
            return 0

        lax.fori_loop(0, RPW, row_body, 0)
        pltpu.sync_copy(outv, out_hbm.at[pl.ds(base_row, RPW)])

    return sc_count


def kernel(query, positive, negative, other_neg):
    B, N, D = query.shape
    BN = B * N
    cpos, cneg = _cosim(query, positive, negative, other_neg)
    RPW = BN // 32
    u1 = _make_sc_count(BN, N, RPW)(cpos, cneg)
    u1 = u1.reshape(B, N)

    n1 = n2 = float(N)
    mu = n1 * n2 / 2.0
    sigma = jnp.sqrt(n1 * n2 * (n1 + n2 + 1.0) / 12.0)
    u2 = n1 * n2 - u1
    z1 = (u1 - mu) / sigma
    z2 = (u2 - mu) / sigma
    inv_sqrt2 = 1.0 / jnp.sqrt(jnp.float32(2.0))
    p1 = 0.5 * (1.0 + jax.scipy.special.erf(z1 * inv_sqrt2))
    p2 = 0.5 * (1.0 + jax.scipy.special.erf(z2 * inv_sqrt2))
    loss = jnp.maximum(1.0 - p2 - 0.05, 0.0) + jnp.maximum(p1 - 0.05, 0.0)
    return jnp.mean(loss)


# speedup vs baseline: 34.4847x; 1.3177x over previous
import functools

import jax
import jax.numpy as jnp
from jax import lax
from jax.experimental import pallas as pl
from jax.experimental.pallas import tpu as pltpu
from jax.experimental.pallas import tpu_sc as plsc


def _cosim_kernel(q_ref, p_ref, n_ref, on_ref, cp_ref, cn_ref):
    eps = 1e-12
    q = q_ref[0]
    on = on_ref[0]
    qv = q - on
    qn = qv / jnp.maximum(
        jnp.sqrt(jnp.sum(qv * qv, axis=1, keepdims=True)), eps)
    pv = p_ref[0] - on
    pn = pv / jnp.maximum(
        jnp.sqrt(jnp.sum(pv * pv, axis=1, keepdims=True)), eps)
    nv = n_ref[0] - on
    nn_ = nv / jnp.maximum(
        jnp.sqrt(jnp.sum(nv * nv, axis=1, keepdims=True)), eps)
    dims = (((1,), (1,)), ((), ()))
    cp_ref[...] = 2.0 - 2.0 * lax.dot_general(
        qn, pn, dims, preferred_element_type=jnp.float32)
    cn_ref[...] = 2.0 - 2.0 * lax.dot_general(
        qn, nn_, dims, preferred_element_type=jnp.float32)


def _cosim(query, positive, negative, other_neg):
    B, N, D = query.shape
    RC = 256
    nblk = N // RC
    grid = (B, nblk)
    return pl.pallas_call(
        _cosim_kernel,
        grid=grid,
        in_specs=[
            pl.BlockSpec((1, RC, D), lambda b, r: (b, r, 0)),
            pl.BlockSpec((1, N, D), lambda b, r: (b, 0, 0)),
            pl.BlockSpec((1, N, D), lambda b, r: (b, 0, 0)),
            pl.BlockSpec((1, 1, D), lambda b, r: (b, 0, 0)),
        ],
        out_specs=[
            pl.BlockSpec((RC, N), lambda b, r, nblk=nblk: (b * nblk + r, 0)),
            pl.BlockSpec((RC, N), lambda b, r, nblk=nblk: (b * nblk + r, 0)),
        ],
        out_shape=[
            jax.ShapeDtypeStruct((B * N, N), jnp.float32),
            jax.ShapeDtypeStruct((B * N, N), jnp.float32),
        ],
    )(query, positive, negative, other_neg)


_V = 16


def _mkey(x):
    bi = lax.bitcast_convert_type(x, jnp.int32)
    sgn = lax.shift_right_arithmetic(bi, 31)
    sgnu = lax.bitcast_convert_type(sgn, jnp.uint32)
    xu = lax.bitcast_convert_type(x, jnp.uint32)
    return xu ^ (sgnu | jnp.uint32(0x80000000))


def _splat(x, dtype=None):
    v = jnp.full((_V,), x)
    return v if dtype is None else v.astype(dtype)


def _perm(v, idx):
    dnums = lax.GatherDimensionNumbers(
        offset_dims=(), collapsed_slice_dims=(0,), start_index_map=(0,))
    return lax.gather(
        v, idx.reshape(_V, 1), dnums, (1,),
        mode=lax.GatherScatterMode.PROMISE_IN_BOUNDS)


def _lane():
    return lax.broadcasted_iota(jnp.int32, (_V,), 0)


def _allreduce(v, op):
    lane = _lane()
    for step in (8, 4, 2, 1):
        v = op(v, _perm(v, lane ^ step))
    return v


def _occurrence(d1):
    lane = _lane()
    occ = jnp.zeros((_V,), jnp.int32)
    one = jnp.ones((_V,), jnp.int32)
    zero = jnp.zeros((_V,), jnp.int32)
    for s in range(1, _V):
        g = _perm(d1, jnp.maximum(lane - s, 0))
        eq = (g == d1) & (lane >= s)
        occ = occ + jnp.where(eq, one, zero)
    return occ


def _occ_rev(d1):
    lane = _lane()
    occ = jnp.zeros((_V,), jnp.int32)
    one = jnp.ones((_V,), jnp.int32)
    zero = jnp.zeros((_V,), jnp.int32)
    for s in range(1, _V):
        g = _perm(d1, jnp.minimum(lane + s, _V - 1))
        eq = (g == d1) & (lane <= _V - 1 - s)
        occ = occ + jnp.where(eq, one, zero)
    return occ


def _prefix_sum(v):
    lane = _lane()
    zero = jnp.zeros((_V,), v.dtype)
    for step in (1, 2, 4, 8):
        g = _perm(v, jnp.maximum(lane - step, 0))
        v = v + jnp.where(lane >= step, g, zero)
    return v


def _make_sc_count(BN, N, RPW):
    VN = N // _V
    mesh = plsc.VectorSubcoreMesh(core_axis_name="c", subcore_axis_name="s")

    @functools.partial(
        pl.kernel, mesh=mesh,
        compiler_params=pltpu.CompilerParams(needs_layout_passes=False),
        out_type=jax.ShapeDtypeStruct((BN,), jnp.float32),
        scratch_types=[
            pltpu.VMEM((N,), jnp.float32),
            pltpu.VMEM((N,), jnp.float32),
            pltpu.VMEM((N,), jnp.int32),
            pltpu.VMEM((N,), jnp.int32),
            pltpu.VMEM((N,), jnp.int32),
            pltpu.VMEM((N,), jnp.int32),
            pltpu.VMEM((N,), jnp.int32),
            pltpu.VMEM((N,), jnp.int32),
            pltpu.VMEM((N,), jnp.int32),
            pltpu.VMEM((RPW,), jnp.float32),
            pltpu.VMEM((_V,), jnp.int32),
        ],
    )
    def sc_count(cp_hbm, cn_hbm, out_hbm, bufP, bufG, keyG, d1G, occG,
                 ctr, ctr2, excl, sortedG, outv, tmpv):
        nc = 2
        wid = lax.axis_index("s") * nc + lax.axis_index("c")
        base_row = wid * RPW
        lane = lax.broadcasted_iota(jnp.int32, (_V,), 0)

        def row_body(r, _):
            row = base_row + r
            pltpu.sync_copy(cp_hbm.at[row], bufP)
            pltpu.sync_copy(cn_hbm.at[row], bufG)

            def mm(v, c):
                mn, mx = c
                a = bufP[pl.ds(v * _V, _V)]
                b = bufG[pl.ds(v * _V, _V)]
                mn = jnp.minimum(jnp.minimum(mn, a), b)
                mx = jnp.maximum(jnp.maximum(mx, a), b)
                return (mn, mx)

            mnv, mxv = lax.fori_loop(
                0, VN, mm,
                (_splat(jnp.float32(jnp.inf)), _splat(-jnp.float32(jnp.inf))))
            mn_v = _mkey(_allreduce(mnv, jnp.minimum))
            mx_v = _mkey(_allreduce(mxv, jnp.maximum))

            rng = (mx_v - mn_v).astype(jnp.float32)
            ebits = lax.shift_right_logical(
                lax.bitcast_convert_type(rng, jnp.int32), 23) & 255
            sh_v = jnp.maximum(ebits - 126 - 11, 0).astype(jnp.uint32)

            def dig(k):
                return lax.shift_right_logical(k - mn_v, sh_v).astype(
                    jnp.int32)

            def zz(v, _c):
                ctr[pl.ds(v * _V, _V)] = jnp.zeros((_V,), jnp.int32)
                ctr2[pl.ds(v * _V, _V)] = jnp.zeros((_V,), jnp.int32)
                return 0

            lax.fori_loop(0, VN, zz, 0)

            def pa(v, _c):
                sl = pl.ds(v * _V, _V)
                k = _mkey(bufG[sl])
                d1 = dig(k)
                keyG[sl] = lax.bitcast_convert_type(k, jnp.int32)
                d1G[sl] = d1
                occ1, lastm = plsc.scan_count(d1)
                occ = occ1 - 1
                occG[sl] = occ + jnp.where(lastm, 256, 0)
                b0 = plsc.load_gather(ctr, [d1])
                plsc.store_scatter(ctr, [d1], b0 + occ + 1, mask=lastm)
                return 0

            lax.fori_loop(0, VN, pa, 0)

            def cs(v, carry):
                sl = pl.ds(v * _V, _V)
                hv = ctr[sl]
                inc = plsc.cumsum(hv)
                excl[sl] = inc - hv + carry
                return carry + _splat(jnp.sum(hv))

            lax.fori_loop(0, VN, cs, jnp.zeros((_V,), jnp.int32))

            def pb(v, _c):
                sl = pl.ds(v * _V, _V)
                d1 = d1G[sl]
                ov = occG[sl]
                occ = ov & 255
                lastm = ov >= 256
                k = keyG[sl]
                b0 = plsc.load_gather(ctr2, [d1])
                e0 = plsc.load_gather(excl, [d1])
                plsc.store_scatter(sortedG, [e0 + b0 + occ], k)
                plsc.store_scatter(ctr2, [d1], b0 + occ + 1, mask=lastm)
                return 0

            lax.fori_loop(0, VN, pb, 0)

            def pc(v, acc):
                sl = pl.ds(v * _V, _V)
                k = _mkey(bufP[sl])
                d1 = dig(k)
                e0 = plsc.load_gather(excl, [d1])
                cnt = plsc.load_gather(ctr, [d1])
                acc = acc + e0
                tmax = _allreduce(cnt, jnp.maximum)[0]
                cm1 = jnp.maximum(cnt - 1, 0)

                def rb(t, a2):
                    tv = _splat(t)
                    g = plsc.load_gather(
                        sortedG, [e0 + jnp.minimum(tv, cm1)])
                    gu = lax.bitcast_convert_type(g, jnp.uint32)
                    hit = (gu < k) & (tv < cnt)
                    return a2 + jnp.where(hit, 1, 0).astype(jnp.int32)

                acc = acc + lax.fori_loop(
                    0, tmax, rb, jnp.zeros((_V,), jnp.int32))
                return acc

            accv = lax.fori_loop(0, VN, pc, jnp.zeros((_V,), jnp.int32))
            u1v = _allreduce(accv, jnp.add).astype(jnp.float32)
            plsc.store_scatter(outv, [_splat(r).astype(jnp.int32)],
                               u1v, mask=lane == 0)
            return 0

        lax.fori_loop(0, RPW, row_body, 0)
        pltpu.sync_copy(outv, out_hbm.at[pl.ds(base_row, RPW)])

    return sc_count


def kernel(query, positive, negative, other_neg):
    B, N, D = query.shape
    BN = B * N
    cpos, cneg = _cosim(query, positive, negative, other_neg)
    RPW = BN // 32
    u1 = _make_sc_count(BN, N, RPW)(cpos, cneg)
    u1 = u1.reshape(B, N)

    n1 = n2 = float(N)
    mu = n1 * n2 / 2.0
    sigma = jnp.sqrt(n1 * n2 * (n1 + n2 + 1.0) / 12.0)
    u2 = n1 * n2 - u1
    z1 = (u1 - mu) / sigma
    z2 = (u2 - mu) / sigma
    inv_sqrt2 = 1.0 / jnp.sqrt(jnp.float32(2.0))
    p1 = 0.5 * (1.0 + jax.scipy.special.erf(z1 * inv_sqrt2))
    p2 = 0.5 * (1.0 + jax.scipy.special.erf(z2 * inv_sqrt2))
    loss = jnp.maximum(1.0 - p2 - 0.05, 0.0) + jnp.maximum(p1 - 0.05, 0.0)
    return jnp.mean(loss)


# speedup vs baseline: 35.6968x; 1.0351x over previous
import functools

import jax
import jax.numpy as jnp
from jax import lax
from jax.experimental import pallas as pl
from jax.experimental.pallas import tpu as pltpu
from jax.experimental.pallas import tpu_sc as plsc


def _cosim_kernel(q_ref, p_ref, n_ref, on_ref, cp_ref, cn_ref):
    eps = 1e-12
    q = q_ref[0]
    on = on_ref[0]
    qv = q - on
    qn = qv / jnp.maximum(
        jnp.sqrt(jnp.sum(qv * qv, axis=1, keepdims=True)), eps)
    pv = p_ref[0] - on
    pn = pv / jnp.maximum(
        jnp.sqrt(jnp.sum(pv * pv, axis=1, keepdims=True)), eps)
    nv = n_ref[0] - on
    nn_ = nv / jnp.maximum(
        jnp.sqrt(jnp.sum(nv * nv, axis=1, keepdims=True)), eps)
    dims = (((1,), (1,)), ((), ()))
    cp_ref[...] = 2.0 - 2.0 * lax.dot_general(
        qn, pn, dims, preferred_element_type=jnp.float32)
    cn_ref[...] = 2.0 - 2.0 * lax.dot_general(
        qn, nn_, dims, preferred_element_type=jnp.float32)


def _cosim(query, positive, negative, other_neg):
    B, N, D = query.shape
    RC = 256
    nblk = N // RC
    grid = (B, nblk)
    return pl.pallas_call(
        _cosim_kernel,
        grid=grid,
        in_specs=[
            pl.BlockSpec((1, RC, D), lambda b, r: (b, r, 0)),
            pl.BlockSpec((1, N, D), lambda b, r: (b, 0, 0)),
            pl.BlockSpec((1, N, D), lambda b, r: (b, 0, 0)),
            pl.BlockSpec((1, 1, D), lambda b, r: (b, 0, 0)),
        ],
        out_specs=[
            pl.BlockSpec((RC, N), lambda b, r, nblk=nblk: (b * nblk + r, 0)),
            pl.BlockSpec((RC, N), lambda b, r, nblk=nblk: (b * nblk + r, 0)),
        ],
        out_shape=[
            jax.ShapeDtypeStruct((B * N, N), jnp.float32),
            jax.ShapeDtypeStruct((B * N, N), jnp.float32),
        ],
    )(query, positive, negative, other_neg)


_V = 16


def _mkey(x):
    bi = lax.bitcast_convert_type(x, jnp.int32)
    sgn = lax.shift_right_arithmetic(bi, 31)
    sgnu = lax.bitcast_convert_type(sgn, jnp.uint32)
    xu = lax.bitcast_convert_type(x, jnp.uint32)
    return xu ^ (sgnu | jnp.uint32(0x80000000))


def _splat(x, dtype=None):
    v = jnp.full((_V,), x)
    return v if dtype is None else v.astype(dtype)


def _perm(v, idx):
    dnums = lax.GatherDimensionNumbers(
        offset_dims=(), collapsed_slice_dims=(0,), start_index_map=(0,))
    return lax.gather(
        v, idx.reshape(_V, 1), dnums, (1,),
        mode=lax.GatherScatterMode.PROMISE_IN_BOUNDS)


def _lane():
    return lax.broadcasted_iota(jnp.int32, (_V,), 0)


def _allreduce(v, op):
    lane = _lane()
    for step in (8, 4, 2, 1):
        v = op(v, _perm(v, lane ^ step))
    return v


def _occurrence(d1):
    lane = _lane()
    occ = jnp.zeros((_V,), jnp.int32)
    one = jnp.ones((_V,), jnp.int32)
    zero = jnp.zeros((_V,), jnp.int32)
    for s in range(1, _V):
        g = _perm(d1, jnp.maximum(lane - s, 0))
        eq = (g == d1) & (lane >= s)
        occ = occ + jnp.where(eq, one, zero)
    return occ


def _occ_rev(d1):
    lane = _lane()
    occ = jnp.zeros((_V,), jnp.int32)
    one = jnp.ones((_V,), jnp.int32)
    zero = jnp.zeros((_V,), jnp.int32)
    for s in range(1, _V):
        g = _perm(d1, jnp.minimum(lane + s, _V - 1))
        eq = (g == d1) & (lane <= _V - 1 - s)
        occ = occ + jnp.where(eq, one, zero)
    return occ


def _prefix_sum(v):
    lane = _lane()
    zero = jnp.zeros((_V,), v.dtype)
    for step in (1, 2, 4, 8):
        g = _perm(v, jnp.maximum(lane - step, 0))
        v = v + jnp.where(lane >= step, g, zero)
    return v


def _make_sc_count(BN, N, RPW):
    VN = N // _V
    mesh = plsc.VectorSubcoreMesh(core_axis_name="c", subcore_axis_name="s")

    @functools.partial(
        pl.kernel, mesh=mesh,
        compiler_params=pltpu.CompilerParams(needs_layout_passes=False),
        out_type=jax.ShapeDtypeStruct((BN,), jnp.float32),
        scratch_types=[
            pltpu.VMEM((N,), jnp.float32),
            pltpu.VMEM((N,), jnp.float32),
            pltpu.VMEM((N,), jnp.int32),
            pltpu.VMEM((N,), jnp.int32),
            pltpu.VMEM((N,), jnp.int32),
            pltpu.VMEM((N,), jnp.int32),
            pltpu.VMEM((N,), jnp.int32),
            pltpu.VMEM((N,), jnp.int32),
            pltpu.VMEM((N,), jnp.int32),
            pltpu.VMEM((RPW,), jnp.float32),
            pltpu.VMEM((_V,), jnp.int32),
        ],
    )
    def sc_count(cp_hbm, cn_hbm, out_hbm, bufP, bufG, keyG, d1G, occG,
                 ctr, ctr2, excl, sortedG, outv, tmpv):
        nc = 2
        wid = lax.axis_index("s") * nc + lax.axis_index("c")
        base_row = wid * RPW
        lane = lax.broadcasted_iota(jnp.int32, (_V,), 0)

        def row_body(r, _):
            row = base_row + r
            pltpu.sync_copy(cp_hbm.at[row], bufP)
            pltpu.sync_copy(cn_hbm.at[row], bufG)

            def mm(v, c):
                mn, mx = c
                a = bufP[pl.ds(v * _V, _V)]
                b = bufG[pl.ds(v * _V, _V)]
                mn = jnp.minimum(jnp.minimum(mn, a), b)
                mx = jnp.maximum(jnp.maximum(mx, a), b)
                return (mn, mx)

            mnv, mxv = lax.fori_loop(
                0, VN, mm,
                (_splat(jnp.float32(jnp.inf)), _splat(-jnp.float32(jnp.inf))))
            mn_v = _mkey(_splat(jnp.min(mnv)))
            mx_v = _mkey(_splat(jnp.max(mxv)))

            rng = (mx_v - mn_v).astype(jnp.float32)
            ebits = lax.shift_right_logical(
                lax.bitcast_convert_type(rng, jnp.int32), 23) & 255
            sh_v = jnp.maximum(ebits - 126 - 11, 0).astype(jnp.uint32)

            def dig(k):
                return lax.shift_right_logical(k - mn_v, sh_v).astype(
                    jnp.int32)

            def zz(v, _c):
                ctr[pl.ds(v * _V, _V)] = jnp.zeros((_V,), jnp.int32)
                ctr2[pl.ds(v * _V, _V)] = jnp.zeros((_V,), jnp.int32)
                return 0

            lax.fori_loop(0, VN, zz, 0)

            def pa(v, _c):
                sl = pl.ds(v * _V, _V)
                k = _mkey(bufG[sl])
                d1 = dig(k)
                keyG[sl] = lax.bitcast_convert_type(k, jnp.int32)
                d1G[sl] = d1
                occ1, lastm = plsc.scan_count(d1)
                occ = occ1 - 1
                occG[sl] = occ + jnp.where(lastm, 256, 0)
                b0 = plsc.load_gather(ctr, [d1])
                plsc.store_scatter(ctr, [d1], b0 + occ + 1, mask=lastm)
                return 0

            lax.fori_loop(0, VN, pa, 0)

            def cs(v, carry):
                sl = pl.ds(v * _V, _V)
                hv = ctr[sl]
                inc = plsc.cumsum(hv)
                excl[sl] = inc - hv + carry
                return carry + _splat(jnp.sum(hv))

            lax.fori_loop(0, VN, cs, jnp.zeros((_V,), jnp.int32))

            def pb(v, _c):
                sl = pl.ds(v * _V, _V)
                d1 = d1G[sl]
                ov = occG[sl]
                occ = ov & 255
                lastm = ov >= 256
                k = keyG[sl]
                b0 = plsc.load_gather(ctr2, [d1])
                e0 = plsc.load_gather(excl, [d1])
                plsc.store_scatter(sortedG, [e0 + b0 + occ], k)
                plsc.store_scatter(ctr2, [d1], b0 + occ + 1, mask=lastm)
                return 0

            lax.fori_loop(0, VN, pb, 0)

            def pc(v, acc):
                sl = pl.ds(v * _V, _V)
                k = _mkey(bufP[sl])
                d1 = dig(k)
                e0 = plsc.load_gather(excl, [d1])
                cnt = plsc.load_gather(ctr, [d1])
                acc = acc + e0
                tmax = jnp.max(cnt)
                cm1 = jnp.maximum(cnt - 1, 0)

                def rb(t, a2):
                    tv = _splat(t)
                    g = plsc.load_gather(
                        sortedG, [e0 + jnp.minimum(tv, cm1)])
                    gu = lax.bitcast_convert_type(g, jnp.uint32)
                    hit = (gu < k) & (tv < cnt)
                    return a2 + jnp.where(hit, 1, 0).astype(jnp.int32)

                acc = acc + lax.fori_loop(
                    0, tmax, rb, jnp.zeros((_V,), jnp.int32))
                return acc

            accv = lax.fori_loop(0, VN, pc, jnp.zeros((_V,), jnp.int32))
            u1v = _splat(jnp.sum(accv)).astype(jnp.float32)
            plsc.store_scatter(outv, [_splat(r).astype(jnp.int32)],
                               u1v, mask=lane == 0)
            return 0

        lax.fori_loop(0, RPW, row_body, 0)
        pltpu.sync_copy(outv, out_hbm.at[pl.ds(base_row, RPW)])

    return sc_count


def kernel(query, positive, negative, other_neg):
    B, N, D = query.shape
    BN = B * N
    cpos, cneg = _cosim(query, positive, negative, other_neg)
    RPW = BN // 32
    u1 = _make_sc_count(BN, N, RPW)(cpos, cneg)
    u1 = u1.reshape(B, N)

    n1 = n2 = float(N)
    mu = n1 * n2 / 2.0
    sigma = jnp.sqrt(n1 * n2 * (n1 + n2 + 1.0) / 12.0)
    u2 = n1 * n2 - u1
    z1 = (u1 - mu) / sigma
    z2 = (u2 - mu) / sigma
    inv_sqrt2 = 1.0 / jnp.sqrt(jnp.float32(2.0))
    p1 = 0.5 * (1.0 + jax.scipy.special.erf(z1 * inv_sqrt2))
    p2 = 0.5 * (1.0 + jax.scipy.special.erf(z2 * inv_sqrt2))
    loss = jnp.maximum(1.0 - p2 - 0.05, 0.0) + jnp.maximum(p1 - 0.05, 0.0)
    return jnp.mean(loss)


# speedup vs baseline: 39.1920x; 1.0979x over previous
import functools

import jax
import jax.numpy as jnp
from jax import lax
from jax.experimental import pallas as pl
from jax.experimental.pallas import tpu as pltpu
from jax.experimental.pallas import tpu_sc as plsc


def _cosim_kernel(q_ref, p_ref, n_ref, on_ref, cp_ref, cn_ref):
    eps = 1e-12
    q = q_ref[0]
    on = on_ref[0]
    qv = q - on
    qn = qv / jnp.maximum(
        jnp.sqrt(jnp.sum(qv * qv, axis=1, keepdims=True)), eps)
    pv = p_ref[0] - on
    pn = pv / jnp.maximum(
        jnp.sqrt(jnp.sum(pv * pv, axis=1, keepdims=True)), eps)
    nv = n_ref[0] - on
    nn_ = nv / jnp.maximum(
        jnp.sqrt(jnp.sum(nv * nv, axis=1, keepdims=True)), eps)
    dims = (((1,), (1,)), ((), ()))
    cp_ref[...] = 2.0 - 2.0 * lax.dot_general(
        qn, pn, dims, preferred_element_type=jnp.float32)
    cn_ref[...] = 2.0 - 2.0 * lax.dot_general(
        qn, nn_, dims, preferred_element_type=jnp.float32)


def _cosim(query, positive, negative, other_neg):
    B, N, D = query.shape
    RC = 256
    nblk = N // RC
    grid = (B, nblk)
    return pl.pallas_call(
        _cosim_kernel,
        grid=grid,
        in_specs=[
            pl.BlockSpec((1, RC, D), lambda b, r: (b, r, 0)),
            pl.BlockSpec((1, N, D), lambda b, r: (b, 0, 0)),
            pl.BlockSpec((1, N, D), lambda b, r: (b, 0, 0)),
            pl.BlockSpec((1, 1, D), lambda b, r: (b, 0, 0)),
        ],
        out_specs=[
            pl.BlockSpec((RC, N), lambda b, r, nblk=nblk: (b * nblk + r, 0)),
            pl.BlockSpec((RC, N), lambda b, r, nblk=nblk: (b * nblk + r, 0)),
        ],
        out_shape=[
            jax.ShapeDtypeStruct((B * N, N), jnp.float32),
            jax.ShapeDtypeStruct((B * N, N), jnp.float32),
        ],
    )(query, positive, negative, other_neg)


_V = 16


def _mkey(x):
    bi = lax.bitcast_convert_type(x, jnp.int32)
    sgn = lax.shift_right_arithmetic(bi, 31)
    sgnu = lax.bitcast_convert_type(sgn, jnp.uint32)
    xu = lax.bitcast_convert_type(x, jnp.uint32)
    return xu ^ (sgnu | jnp.uint32(0x80000000))


def _splat(x, dtype=None):
    v = jnp.full((_V,), x)
    return v if dtype is None else v.astype(dtype)


def _perm(v, idx):
    dnums = lax.GatherDimensionNumbers(
        offset_dims=(), collapsed_slice_dims=(0,), start_index_map=(0,))
    return lax.gather(
        v, idx.reshape(_V, 1), dnums, (1,),
        mode=lax.GatherScatterMode.PROMISE_IN_BOUNDS)


def _lane():
    return lax.broadcasted_iota(jnp.int32, (_V,), 0)


def _allreduce(v, op):
    lane = _lane()
    for step in (8, 4, 2, 1):
        v = op(v, _perm(v, lane ^ step))
    return v


def _occurrence(d1):
    lane = _lane()
    occ = jnp.zeros((_V,), jnp.int32)
    one = jnp.ones((_V,), jnp.int32)
    zero = jnp.zeros((_V,), jnp.int32)
    for s in range(1, _V):
        g = _perm(d1, jnp.maximum(lane - s, 0))
        eq = (g == d1) & (lane >= s)
        occ = occ + jnp.where(eq, one, zero)
    return occ


def _occ_rev(d1):
    lane = _lane()
    occ = jnp.zeros((_V,), jnp.int32)
    one = jnp.ones((_V,), jnp.int32)
    zero = jnp.zeros((_V,), jnp.int32)
    for s in range(1, _V):
        g = _perm(d1, jnp.minimum(lane + s, _V - 1))
        eq = (g == d1) & (lane <= _V - 1 - s)
        occ = occ + jnp.where(eq, one, zero)
    return occ


def _prefix_sum(v):
    lane = _lane()
    zero = jnp.zeros((_V,), v.dtype)
    for step in (1, 2, 4, 8):
        g = _perm(v, jnp.maximum(lane - step, 0))
        v = v + jnp.where(lane >= step, g, zero)
    return v


def _make_sc_count(BN, N, RPW):
    VN = N // _V
    NB = 2 * N
    VB = NB // _V
    mesh = plsc.VectorSubcoreMesh(core_axis_name="c", subcore_axis_name="s")

    @functools.partial(
        pl.kernel, mesh=mesh,
        compiler_params=pltpu.CompilerParams(needs_layout_passes=False),
        out_type=jax.ShapeDtypeStruct((BN,), jnp.float32),
        scratch_types=[
            pltpu.VMEM((N,), jnp.float32),
            pltpu.VMEM((N,), jnp.float32),
            pltpu.VMEM((N,), jnp.int32),
            pltpu.VMEM((N,), jnp.int32),
            pltpu.VMEM((N,), jnp.int32),
            pltpu.VMEM((2 * N,), jnp.int32),
            pltpu.VMEM((2 * N,), jnp.int32),
            pltpu.VMEM((2 * N,), jnp.int32),
            pltpu.VMEM((N,), jnp.int32),
            pltpu.VMEM((RPW,), jnp.float32),
            pltpu.VMEM((_V,), jnp.int32),
        ],
    )
    def sc_count(cp_hbm, cn_hbm, out_hbm, bufP, bufG, keyG, d1G, occG,
                 ctr, ctr2, excl, sortedG, outv, tmpv):
        nc = 2
        wid = lax.axis_index("s") * nc + lax.axis_index("c")
        base_row = wid * RPW
        lane = lax.broadcasted_iota(jnp.int32, (_V,), 0)

        def row_body(r, _):
            row = base_row + r
            pltpu.sync_copy(cp_hbm.at[row], bufP)
            pltpu.sync_copy(cn_hbm.at[row], bufG)

            def mm(v, c):
                mn, mx = c
                a = bufP[pl.ds(v * _V, _V)]
                b = bufG[pl.ds(v * _V, _V)]
                mn = jnp.minimum(jnp.minimum(mn, a), b)
                mx = jnp.maximum(jnp.maximum(mx, a), b)
                return (mn, mx)

            mnv, mxv = lax.fori_loop(
                0, VN, mm,
                (_splat(jnp.float32(jnp.inf)), _splat(-jnp.float32(jnp.inf))))
            mn_v = _mkey(_splat(jnp.min(mnv)))
            mx_v = _mkey(_splat(jnp.max(mxv)))

            rng = (mx_v - mn_v).astype(jnp.float32)
            ebits = lax.shift_right_logical(
                lax.bitcast_convert_type(rng, jnp.int32), 23) & 255
            sh_v = jnp.maximum(ebits - 126 - 12, 0).astype(jnp.uint32)

            def dig(k):
                return lax.shift_right_logical(k - mn_v, sh_v).astype(
                    jnp.int32)

            def zz(v, _c):
                ctr[pl.ds(v * _V, _V)] = jnp.zeros((_V,), jnp.int32)
                ctr2[pl.ds(v * _V, _V)] = jnp.zeros((_V,), jnp.int32)
                return 0

            lax.fori_loop(0, VB, zz, 0)

            def pa(v, _c):
                sl = pl.ds(v * _V, _V)
                k = _mkey(bufG[sl])
                d1 = dig(k)
                keyG[sl] = lax.bitcast_convert_type(k, jnp.int32)
                d1G[sl] = d1
                occ1, lastm = plsc.scan_count(d1)
                occ = occ1 - 1
                occG[sl] = occ + jnp.where(lastm, 256, 0)
                b0 = plsc.load_gather(ctr, [d1])
                plsc.store_scatter(ctr, [d1], b0 + occ + 1, mask=lastm)
                return 0

            lax.fori_loop(0, VN, pa, 0)

            def cs(v, carry):
                sl = pl.ds(v * _V, _V)
                hv = ctr[sl]
                inc = plsc.cumsum(hv)
                excl[sl] = inc - hv + carry
                return carry + _splat(jnp.sum(hv))

            lax.fori_loop(0, VB, cs, jnp.zeros((_V,), jnp.int32))

            def pb(v, _c):
                sl = pl.ds(v * _V, _V)
                d1 = d1G[sl]
                ov = occG[sl]
                occ = ov & 255
                lastm = ov >= 256
                k = keyG[sl]
                b0 = plsc.load_gather(ctr2, [d1])
                e0 = plsc.load_gather(excl, [d1])
                plsc.store_scatter(sortedG, [e0 + b0 + occ], k)
                plsc.store_scatter(ctr2, [d1], b0 + occ + 1, mask=lastm)
                return 0

            lax.fori_loop(0, VN, pb, 0)

            def pc(v, acc):
                sl = pl.ds(v * _V, _V)
                k = _mkey(bufP[sl])
                d1 = dig(k)
                e0 = plsc.load_gather(excl, [d1])
                cnt = plsc.load_gather(ctr, [d1])
                acc = acc + e0
                tmax = jnp.max(cnt)
                cm1 = jnp.maximum(cnt - 1, 0)

                def rb(t, a2):
                    for u in range(2):
                        tv = _splat(t * 2 + u)
                        g = plsc.load_gather(
                            sortedG, [e0 + jnp.minimum(tv, cm1)])
                        gu = lax.bitcast_convert_type(g, jnp.uint32)
                        hit = (gu < k) & (tv < cnt)
                        a2 = a2 + jnp.where(hit, 1, 0).astype(jnp.int32)
                    return a2

                acc = acc + lax.fori_loop(
                    0, (tmax + 1) // 2, rb, jnp.zeros((_V,), jnp.int32))
                return acc

            accv = lax.fori_loop(0, VN, pc, jnp.zeros((_V,), jnp.int32))
            u1v = _splat(jnp.sum(accv)).astype(jnp.float32)
            plsc.store_scatter(outv, [_splat(r).astype(jnp.int32)],
                               u1v, mask=lane == 0)
            return 0

        lax.fori_loop(0, RPW, row_body, 0)
        pltpu.sync_copy(outv, out_hbm.at[pl.ds(base_row, RPW)])

    return sc_count


def kernel(query, positive, negative, other_neg):
    B, N, D = query.shape
    BN = B * N
    cpos, cneg = _cosim(query, positive, negative, other_neg)
    RPW = BN // 32
    u1 = _make_sc_count(BN, N, RPW)(cpos, cneg)
    u1 = u1.reshape(B, N)

    n1 = n2 = float(N)
    mu = n1 * n2 / 2.0
    sigma = jnp.sqrt(n1 * n2 * (n1 + n2 + 1.0) / 12.0)
    u2 = n1 * n2 - u1
    z1 = (u1 - mu) / sigma
    z2 = (u2 - mu) / sigma
    inv_sqrt2 = 1.0 / jnp.sqrt(jnp.float32(2.0))
    p1 = 0.5 * (1.0 + jax.scipy.special.erf(z1 * inv_sqrt2))
    p2 = 0.5 * (1.0 + jax.scipy.special.erf(z2 * inv_sqrt2))
    loss = jnp.maximum(1.0 - p2 - 0.05, 0.0) + jnp.maximum(p1 - 0.05, 0.0)
    return jnp.mean(loss)


# speedup vs baseline: 42.3475x; 1.0805x over previous
import functools

import jax
import jax.numpy as jnp
from jax import lax
from jax.experimental import pallas as pl
from jax.experimental.pallas import tpu as pltpu
from jax.experimental.pallas import tpu_sc as plsc


def _cosim_kernel(q_ref, p_ref, n_ref, on_ref, cp_ref, cn_ref):
    eps = 1e-12
    q = q_ref[0]
    on = on_ref[0]
    qv = q - on
    qn = qv / jnp.maximum(
        jnp.sqrt(jnp.sum(qv * qv, axis=1, keepdims=True)), eps)
    pv = p_ref[0] - on
    pn = pv / jnp.maximum(
        jnp.sqrt(jnp.sum(pv * pv, axis=1, keepdims=True)), eps)
    nv = n_ref[0] - on
    nn_ = nv / jnp.maximum(
        jnp.sqrt(jnp.sum(nv * nv, axis=1, keepdims=True)), eps)
    dims = (((1,), (1,)), ((), ()))
    cp_ref[...] = 2.0 - 2.0 * lax.dot_general(
        qn, pn, dims, preferred_element_type=jnp.float32)
    cn_ref[...] = 2.0 - 2.0 * lax.dot_general(
        qn, nn_, dims, preferred_element_type=jnp.float32)


def _cosim(query, positive, negative, other_neg):
    B, N, D = query.shape
    RC = 256
    nblk = N // RC
    grid = (B, nblk)
    return pl.pallas_call(
        _cosim_kernel,
        grid=grid,
        in_specs=[
            pl.BlockSpec((1, RC, D), lambda b, r: (b, r, 0)),
            pl.BlockSpec((1, N, D), lambda b, r: (b, 0, 0)),
            pl.BlockSpec((1, N, D), lambda b, r: (b, 0, 0)),
            pl.BlockSpec((1, 1, D), lambda b, r: (b, 0, 0)),
        ],
        out_specs=[
            pl.BlockSpec((RC, N), lambda b, r, nblk=nblk: (b * nblk + r, 0)),
            pl.BlockSpec((RC, N), lambda b, r, nblk=nblk: (b * nblk + r, 0)),
        ],
        out_shape=[
            jax.ShapeDtypeStruct((B * N, N), jnp.float32),
            jax.ShapeDtypeStruct((B * N, N), jnp.float32),
        ],
    )(query, positive, negative, other_neg)


_V = 16


def _mkey(x):
    bi = lax.bitcast_convert_type(x, jnp.int32)
    sgn = lax.shift_right_arithmetic(bi, 31)
    sgnu = lax.bitcast_convert_type(sgn, jnp.uint32)
    xu = lax.bitcast_convert_type(x, jnp.uint32)
    return xu ^ (sgnu | jnp.uint32(0x80000000))


def _splat(x, dtype=None):
    v = jnp.full((_V,), x)
    return v if dtype is None else v.astype(dtype)


def _perm(v, idx):
    dnums = lax.GatherDimensionNumbers(
        offset_dims=(), collapsed_slice_dims=(0,), start_index_map=(0,))
    return lax.gather(
        v, idx.reshape(_V, 1), dnums, (1,),
        mode=lax.GatherScatterMode.PROMISE_IN_BOUNDS)


def _lane():
    return lax.broadcasted_iota(jnp.int32, (_V,), 0)


def _allreduce(v, op):
    lane = _lane()
    for step in (8, 4, 2, 1):
        v = op(v, _perm(v, lane ^ step))
    return v


def _occurrence(d1):
    lane = _lane()
    occ = jnp.zeros((_V,), jnp.int32)
    one = jnp.ones((_V,), jnp.int32)
    zero = jnp.zeros((_V,), jnp.int32)
    for s in range(1, _V):
        g = _perm(d1, jnp.maximum(lane - s, 0))
        eq = (g == d1) & (lane >= s)
        occ = occ + jnp.where(eq, one, zero)
    return occ


def _occ_rev(d1):
    lane = _lane()
    occ = jnp.zeros((_V,), jnp.int32)
    one = jnp.ones((_V,), jnp.int32)
    zero = jnp.zeros((_V,), jnp.int32)
    for s in range(1, _V):
        g = _perm(d1, jnp.minimum(lane + s, _V - 1))
        eq = (g == d1) & (lane <= _V - 1 - s)
        occ = occ + jnp.where(eq, one, zero)
    return occ


def _prefix_sum(v):
    lane = _lane()
    zero = jnp.zeros((_V,), v.dtype)
    for step in (1, 2, 4, 8):
        g = _perm(v, jnp.maximum(lane - step, 0))
        v = v + jnp.where(lane >= step, g, zero)
    return v


def _make_sc_count(BN, N, RPW):
    VN = N // _V
    NB = 2 * N
    VB = NB // _V
    mesh = plsc.VectorSubcoreMesh(core_axis_name="c", subcore_axis_name="s")

    @functools.partial(
        pl.kernel, mesh=mesh,
        compiler_params=pltpu.CompilerParams(needs_layout_passes=False),
        out_type=jax.ShapeDtypeStruct((BN,), jnp.float32),
        scratch_types=[
            pltpu.VMEM((N,), jnp.float32),
            pltpu.VMEM((N,), jnp.float32),
            pltpu.VMEM((N,), jnp.int32),
            pltpu.VMEM((N,), jnp.int32),
            pltpu.VMEM((N,), jnp.int32),
            pltpu.VMEM((2 * N,), jnp.int32),
            pltpu.VMEM((2 * N,), jnp.int32),
            pltpu.VMEM((2 * N,), jnp.int32),
            pltpu.VMEM((N,), jnp.int32),
            pltpu.VMEM((RPW,), jnp.float32),
            pltpu.VMEM((_V,), jnp.int32),
        ],
    )
    def sc_count(cp_hbm, cn_hbm, out_hbm, bufP, bufG, keyG, d1G, occG,
                 ctr, ctr2, excl, sortedG, outv, tmpv):
        nc = 2
        wid = lax.axis_index("s") * nc + lax.axis_index("c")
        base_row = wid * RPW
        lane = lax.broadcasted_iota(jnp.int32, (_V,), 0)

        def row_body(r, _):
            row = base_row + r
            pltpu.sync_copy(cp_hbm.at[row], bufP)
            pltpu.sync_copy(cn_hbm.at[row], bufG)

            def mm(v, c):
                mn, mx = c
                for u in range(2):
                    sl = pl.ds((v * 2 + u) * _V, _V)
                    a = bufP[sl]
                    b = bufG[sl]
                    mn = jnp.minimum(jnp.minimum(mn, a), b)
                    mx = jnp.maximum(jnp.maximum(mx, a), b)
                return (mn, mx)

            mnv, mxv = lax.fori_loop(
                0, VN // 2, mm,
                (_splat(jnp.float32(jnp.inf)), _splat(-jnp.float32(jnp.inf))))
            mn_v = _mkey(_splat(jnp.min(mnv)))
            mx_v = _mkey(_splat(jnp.max(mxv)))

            rng = (mx_v - mn_v).astype(jnp.float32)
            ebits = lax.shift_right_logical(
                lax.bitcast_convert_type(rng, jnp.int32), 23) & 255
            sh_v = jnp.maximum(ebits - 126 - 12, 0).astype(jnp.uint32)

            def dig(k):
                return lax.shift_right_logical(k - mn_v, sh_v).astype(
                    jnp.int32)

            def zz(v, _c):
                for u in range(2):
                    sl = pl.ds((v * 2 + u) * _V, _V)
                    ctr[sl] = jnp.zeros((_V,), jnp.int32)
                    ctr2[sl] = jnp.zeros((_V,), jnp.int32)
                return 0

            lax.fori_loop(0, VB // 2, zz, 0)

            def pa(v, _c):
                for u in range(2):
                    sl = pl.ds((v * 2 + u) * _V, _V)
                    k = _mkey(bufG[sl])
                    d1 = dig(k)
                    keyG[sl] = lax.bitcast_convert_type(k, jnp.int32)
                    d1G[sl] = d1
                    occ1, lastm = plsc.scan_count(d1)
                    occ = occ1 - 1
                    occG[sl] = occ + jnp.where(lastm, 256, 0)
                    b0 = plsc.load_gather(ctr, [d1])
                    plsc.store_scatter(ctr, [d1], b0 + occ + 1, mask=lastm)
                return 0

            lax.fori_loop(0, VN // 2, pa, 0)

            def cs(v, carry):
                for u in range(2):
                    sl = pl.ds((v * 2 + u) * _V, _V)
                    hv = ctr[sl]
                    inc = plsc.cumsum(hv)
                    excl[sl] = inc - hv + carry
                    carry = carry + _splat(jnp.sum(hv))
                return carry

            lax.fori_loop(0, VB // 2, cs, jnp.zeros((_V,), jnp.int32))

            def pb(v, _c):
                for u in range(2):
                    sl = pl.ds((v * 2 + u) * _V, _V)
                    d1 = d1G[sl]
                    ov = occG[sl]
                    occ = ov & 255
                    lastm = ov >= 256
                    k = keyG[sl]
                    b0 = plsc.load_gather(ctr2, [d1])
                    e0 = plsc.load_gather(excl, [d1])
                    plsc.store_scatter(sortedG, [e0 + b0 + occ], k)
                    plsc.store_scatter(ctr2, [d1], b0 + occ + 1, mask=lastm)
                return 0

            lax.fori_loop(0, VN // 2, pb, 0)

            def pc(v, acc):
                sl = pl.ds(v * _V, _V)
                k = _mkey(bufP[sl])
                d1 = dig(k)
                e0 = plsc.load_gather(excl, [d1])
                cnt = plsc.load_gather(ctr, [d1])
                acc = acc + e0
                tmax = jnp.max(cnt)
                cm1 = jnp.maximum(cnt - 1, 0)

                def rb(t, a2):
                    for u in range(2):
                        tv = _splat(t * 2 + u)
                        g = plsc.load_gather(
                            sortedG, [e0 + jnp.minimum(tv, cm1)])
                        gu = lax.bitcast_convert_type(g, jnp.uint32)
                        hit = (gu < k) & (tv < cnt)
                        a2 = a2 + jnp.where(hit, 1, 0).astype(jnp.int32)
                    return a2

                acc = acc + lax.fori_loop(
                    0, (tmax + 1) // 2, rb, jnp.zeros((_V,), jnp.int32))
                return acc

            accv = lax.fori_loop(0, VN, pc, jnp.zeros((_V,), jnp.int32))
            u1v = _splat(jnp.sum(accv)).astype(jnp.float32)
            plsc.store_scatter(outv, [_splat(r).astype(jnp.int32)],
                               u1v, mask=lane == 0)
            return 0

        lax.fori_loop(0, RPW, row_body, 0)
        pltpu.sync_copy(outv, out_hbm.at[pl.ds(base_row, RPW)])

    return sc_count


def kernel(query, positive, negative, other_neg):
    B, N, D = query.shape
    BN = B * N
    cpos, cneg = _cosim(query, positive, negative, other_neg)
    RPW = BN // 32
    u1 = _make_sc_count(BN, N, RPW)(cpos, cneg)
    u1 = u1.reshape(B, N)

    n1 = n2 = float(N)
    mu = n1 * n2 / 2.0
    sigma = jnp.sqrt(n1 * n2 * (n1 + n2 + 1.0) / 12.0)
    u2 = n1 * n2 - u1
    z1 = (u1 - mu) / sigma
    z2 = (u2 - mu) / sigma
    inv_sqrt2 = 1.0 / jnp.sqrt(jnp.float32(2.0))
    p1 = 0.5 * (1.0 + jax.scipy.special.erf(z1 * inv_sqrt2))
    p2 = 0.5 * (1.0 + jax.scipy.special.erf(z2 * inv_sqrt2))
    loss = jnp.maximum(1.0 - p2 - 0.05, 0.0) + jnp.maximum(p1 - 0.05, 0.0)
    return jnp.mean(loss)
